# baseline (speedup 1.0000x reference)
import functools

import jax
import jax.numpy as jnp
from jax import lax
from jax.experimental import pallas as pl
from jax.experimental.pallas import tpu as pltpu
from jax.experimental.pallas import tpu_sc as plsc

_CH = 40
_NW = 32
_EV = 16


def _sc_body(B, D, E, clr_hbm, otu_hbm, w_hbm, b_hbm, out_hbm,
             tab_v, clr_v, w_v, b_v, out_buf, sems):
    nchunks = D // _CH
    wid = lax.axis_index("s") * 2 + lax.axis_index("c")

    pltpu.sync_copy(w_hbm, w_v)
    pltpu.sync_copy(b_hbm, b_v)
    w_regs = [w_v[pl.ds(ev * _EV, _EV)] for ev in range(E // _EV)]
    b_regs = [b_v[pl.ds(ev * _EV, _EV)] for ev in range(E // _EV)]

    nk = (nchunks - wid + _NW - 1) // _NW

    def chunk_body(k, _):
        ch = wid + k * _NW
        row0 = ch * _CH
        pltpu.sync_copy(otu_hbm.at[pl.ds(row0, _CH), :], tab_v)
        pltpu.sync_copy(clr_hbm.at[pl.ds(row0, _CH), :], clr_v)

        def fold_row(r, _):
            for ev in range(E // _EV):
                sl = pl.ds(ev * _EV, _EV)
                tab_v[r, sl] = tab_v[r, sl] + b_regs[ev]
            return 0

        lax.fori_loop(0, _CH, fold_row, 0)

        def batch_body(b, _):
            slot = lax.rem(b, 2)

            @pl.when(b >= 2)
            def _wait_prev():
                pltpu.make_async_copy(
                    out_buf.at[slot], out_hbm.at[0, pl.ds(0, _CH), :],
                    sems.at[slot]).wait()

            def row_body(r, _):
                bvec0 = clr_v[r, pl.ds(0, _EV)]
                bvec1 = clr_v[r, pl.ds(_EV, _EV)]
                idx = jnp.full((_EV, 1), lax.rem(b, _EV), jnp.int32)
                dn = lax.GatherDimensionNumbers(
                    offset_dims=(), collapsed_slice_dims=(0,),
                    start_index_map=(0,))
                g0 = lax.gather(bvec0, idx, dn, (1,),
                                mode=lax.GatherScatterMode.PROMISE_IN_BOUNDS)
                g1 = lax.gather(bvec1, idx, dn, (1,),
                                mode=lax.GatherScatterMode.PROMISE_IN_BOUNDS)
                csplat = jnp.where(b < _EV, g0, g1)
                for ev in range(E // _EV):
                    sl = pl.ds(ev * _EV, _EV)
                    out_buf[slot, r, sl] = tab_v[r, sl] + csplat * w_regs[ev]
                return 0

            lax.fori_loop(0, _CH, row_body, 0)
            pltpu.make_async_copy(
                out_buf.at[slot], out_hbm.at[b, pl.ds(row0, _CH), :],
                sems.at[slot]).start()
            return 0

        lax.fori_loop(0, B, batch_body, 0)
        for slot in range(2):
            pltpu.make_async_copy(
                out_buf.at[slot], out_hbm.at[0, pl.ds(0, _CH), :],
                sems.at[slot]).wait()
        return 0

    lax.fori_loop(0, nk, chunk_body, 0)


def kernel(clr, otu_table, W_val, b_val):
    B, D = clr.shape
    E = otu_table.shape[1]

    mesh = plsc.VectorSubcoreMesh(core_axis_name="c", subcore_axis_name="s")
    sc_kernel = functools.partial(
        pl.kernel,
        mesh=mesh,
        out_type=jax.ShapeDtypeStruct((B, D, E), jnp.float32),
        scratch_types=[
            pltpu.VMEM((_CH, E), jnp.float32),
            pltpu.VMEM((_CH, B), jnp.float32),
            pltpu.VMEM((E,), jnp.float32),
            pltpu.VMEM((E,), jnp.float32),
            pltpu.VMEM((2, _CH, E), jnp.float32),
            pltpu.SemaphoreType.DMA((2,)),
        ],
    )(functools.partial(_sc_body, B, D, E))
    return sc_kernel(clr.T, otu_table, W_val[:, 0].reshape(E), b_val)


# speedup vs baseline: 7.2382x; 7.2382x over previous
import functools

import jax
import jax.numpy as jnp
from jax.experimental import pallas as pl
from jax.experimental.pallas import tpu as pltpu

_BBLK = 16
_NBUF = 2


def _body(nbb, nsteps, otu_ref, clr_ref, sel_ref, w_ref, b_ref, out_ref,
          buf_ref, tpb_ref, sems):
    i = pl.program_id(0)
    bblk = buf_ref.shape[1]
    dblk = buf_ref.shape[2]
    slot = jax.lax.rem(i, _NBUF)
    g_idx = jax.lax.rem(i, nbb)

    dst0 = out_ref.at[0, pl.ds(0, dblk), :]

    @pl.when(i >= _NBUF)
    def _wait_prev():
        for j in range(bblk):
            pltpu.make_async_copy(buf_ref.at[slot, j], dst0, sems.at[slot]).wait()

    @pl.when(g_idx == 0)
    def _fold_bias():
        tpb_ref[...] = otu_ref[...] + b_ref[...]

    blk = clr_ref[0]
    cols = jnp.dot(
        blk, sel_ref[0],
        preferred_element_type=jnp.float32,
        precision=jax.lax.Precision.HIGHEST,
    )
    for j in range(bblk):
        buf_ref[slot, j] = tpb_ref[...] + cols[:, j:j + 1] * w_ref[...]

    d_idx = i // nbb
    for j in range(bblk):
        dst = out_ref.at[g_idx * bblk + j, pl.ds(d_idx * dblk, dblk), :]
        pltpu.make_async_copy(buf_ref.at[slot, j], dst, sems.at[slot]).start()

    @pl.when(i == nsteps - 1)
    def _drain():
        for k in range(_NBUF):
            for j in range(bblk):
                pltpu.make_async_copy(buf_ref.at[k, j], dst0, sems.at[k]).wait()


def _pick_dblk(d: int) -> int:
    best = 8
    for cand in range(8, 1025, 8):
        if d % cand == 0:
            best = cand
    return best


def kernel(clr, otu_table, W_val, b_val):
    B, D = clr.shape
    E = otu_table.shape[1]
    dblk = _pick_dblk(D)
    ndb = D // dblk
    bblk = _BBLK if B % _BBLK == 0 else 1
    nbb = B // bblk
    nsteps = ndb * nbb

    clr3 = clr.T.reshape(ndb, dblk, B)
    w_row = W_val[:, 0].reshape(1, E)
    b_row = b_val.reshape(1, E)
    sel3 = (
        jax.lax.broadcasted_iota(jnp.int32, (nbb, B, bblk), 1)
        == jax.lax.broadcasted_iota(jnp.int32, (nbb, B, bblk), 2)
        + jax.lax.broadcasted_iota(jnp.int32, (nbb, B, bblk), 0) * bblk
    ).astype(jnp.float32)

    out = pl.pallas_call(
        functools.partial(_body, nbb, nsteps),
        grid=(nsteps,),
        in_specs=[
            pl.BlockSpec((dblk, E), lambda i: (i // nbb, 0)),
            pl.BlockSpec((1, dblk, B), lambda i: (i // nbb, 0, 0)),
            pl.BlockSpec((1, B, bblk), lambda i: (i % nbb, 0, 0)),
            pl.BlockSpec((1, E), lambda i: (0, 0)),
            pl.BlockSpec((1, E), lambda i: (0, 0)),
        ],
        out_specs=pl.BlockSpec(memory_space=pltpu.MemorySpace.HBM),
        out_shape=jax.ShapeDtypeStruct((B, D, E), jnp.float32),
        scratch_shapes=[
            pltpu.VMEM((_NBUF, bblk, dblk, E), jnp.float32),
            pltpu.VMEM((dblk, E), jnp.float32),
            pltpu.SemaphoreType.DMA((_NBUF,)),
        ],
    )(otu_table, clr3, sel3, w_row, b_row)
    return out


# speedup vs baseline: 7.5023x; 1.0365x over previous
import functools

import jax
import jax.numpy as jnp
from jax.experimental import pallas as pl
from jax.experimental.pallas import tpu as pltpu

_BBLK = 32
_NBUF = 2


def _body(nbb, nsteps, otu_ref, clr_ref, sel_ref, w_ref, b_ref, out_ref,
          buf_ref, tpb_ref, sems):
    i = pl.program_id(0)
    bblk = buf_ref.shape[1]
    dblk = buf_ref.shape[2]
    slot = jax.lax.rem(i, _NBUF)
    g_idx = jax.lax.rem(i, nbb)

    dst0 = out_ref.at[0, pl.ds(0, dblk), :]

    @pl.when(i >= _NBUF)
    def _wait_prev():
        for j in range(bblk):
            pltpu.make_async_copy(buf_ref.at[slot, j], dst0, sems.at[slot]).wait()

    @pl.when(g_idx == 0)
    def _fold_bias():
        tpb_ref[...] = otu_ref[...] + b_ref[...]

    blk = clr_ref[0]
    if bblk == blk.shape[1]:
        cols = blk
    else:
        cols = jnp.dot(
            blk, sel_ref[0],
            preferred_element_type=jnp.float32,
            precision=jax.lax.Precision.HIGHEST,
        )
    for j in range(bblk):
        buf_ref[slot, j] = tpb_ref[...] + cols[:, j:j + 1] * w_ref[...]

    d_idx = i // nbb
    for j in range(bblk):
        dst = out_ref.at[g_idx * bblk + j, pl.ds(d_idx * dblk, dblk), :]
        pltpu.make_async_copy(buf_ref.at[slot, j], dst, sems.at[slot]).start()

    @pl.when(i == nsteps - 1)
    def _drain():
        for k in range(_NBUF):
            for j in range(bblk):
                pltpu.make_async_copy(buf_ref.at[k, j], dst0, sems.at[k]).wait()


def _pick_dblk(d: int) -> int:
    best = 8
    for cand in range(8, 513, 8):
        if d % cand == 0:
            best = cand
    return best


def kernel(clr, otu_table, W_val, b_val):
    B, D = clr.shape
    E = otu_table.shape[1]
    dblk = _pick_dblk(D)
    ndb = D // dblk
    bblk = _BBLK if B % _BBLK == 0 else 1
    nbb = B // bblk
    nsteps = ndb * nbb

    clr3 = clr.T.reshape(ndb, dblk, B)
    w_row = W_val[:, 0].reshape(1, E)
    b_row = b_val.reshape(1, E)
    sel3 = (
        jax.lax.broadcasted_iota(jnp.int32, (nbb, B, bblk), 1)
        == jax.lax.broadcasted_iota(jnp.int32, (nbb, B, bblk), 2)
        + jax.lax.broadcasted_iota(jnp.int32, (nbb, B, bblk), 0) * bblk
    ).astype(jnp.float32)

    out = pl.pallas_call(
        functools.partial(_body, nbb, nsteps),
        grid=(nsteps,),
        in_specs=[
            pl.BlockSpec((dblk, E), lambda i: (i // nbb, 0)),
            pl.BlockSpec((1, dblk, B), lambda i: (i // nbb, 0, 0)),
            pl.BlockSpec((1, B, bblk), lambda i: (i % nbb, 0, 0)),
            pl.BlockSpec((1, E), lambda i: (0, 0)),
            pl.BlockSpec((1, E), lambda i: (0, 0)),
        ],
        out_specs=pl.BlockSpec(memory_space=pltpu.MemorySpace.HBM),
        out_shape=jax.ShapeDtypeStruct((B, D, E), jnp.float32),
        scratch_shapes=[
            pltpu.VMEM((_NBUF, bblk, dblk, E), jnp.float32),
            pltpu.VMEM((dblk, E), jnp.float32),
            pltpu.SemaphoreType.DMA((_NBUF,)),
        ],
    )(otu_table, clr3, sel3, w_row, b_row)
    return out
